# baseline (speedup 1.0000x reference)
import functools

import jax
import jax.numpy as jnp
from jax import lax
from jax.experimental import pallas as pl
from jax.experimental.pallas import tpu as pltpu
from jax.experimental.pallas import tpu_sc as plsc

NBINS = 512
XLO = -9.0
XHI = 9.0
SCALE = NBINS / (XHI - XLO)
NLANES = 16
NSLOTS = 2 * NBINS * NLANES
TSHIFT = (NBINS * NLANES).bit_length() - 1
HR = NSLOTS // 128
NEGR = HR // 2

IMG = 16
ROWS = 512
COLS = 512

ROWS_PER_CHUNK = 32
CHUNK = ROWS_PER_CHUNK * COLS
HALF = ROWS // 2
NCHUNKS = HALF // ROWS_PER_CHUNK


def _tc_prepass_body(x_ref, t_ref, s_ref):
    s16 = SCALE * 16.0
    off16 = -XLO * SCALE * 16.0
    hi16 = float((NBINS - 1) * 16 + 15)
    x = x_ref[...]
    t = t_ref[...]
    u = x * s16 + off16
    u = jnp.minimum(jnp.maximum(u, 0.0), hi16)
    b16 = jnp.bitwise_and(u.astype(jnp.int32), ~15)
    tshift = jnp.left_shift(t.astype(jnp.int32), TSHIFT)
    lane = jnp.bitwise_and(
        lax.broadcasted_iota(jnp.int32, x.shape, x.ndim - 1), 15)
    slots = b16 + tshift + lane
    lo = slots[:, :, : COLS // 2]
    hi = slots[:, :, COLS // 2 :]
    s_ref[...] = jnp.bitwise_or(lo, jnp.left_shift(hi, 16))


PRE_SPLIT = 1
PRE_R = ROWS // PRE_SPLIT


def _tc_prepass(x3, t3):
    return pl.pallas_call(
        _tc_prepass_body,
        grid=(IMG * PRE_SPLIT,),
        in_specs=[
            pl.BlockSpec((1, PRE_R, COLS),
                         lambda i: (i // PRE_SPLIT, i % PRE_SPLIT, 0)),
            pl.BlockSpec((1, PRE_R, COLS),
                         lambda i: (i // PRE_SPLIT, i % PRE_SPLIT, 0)),
        ],
        out_specs=pl.BlockSpec((1, PRE_R, COLS // 2),
                               lambda i: (i // PRE_SPLIT, i % PRE_SPLIT, 0)),
        out_shape=jax.ShapeDtypeStruct((IMG, ROWS, COLS // 2), jnp.int32),
    )(x3, t3)


def _sc_histogram(slots3):
    info = plsc.get_sparse_core_info()
    nc, ns = info.num_cores, info.num_subcores
    nw = nc * ns
    mesh = plsc.VectorSubcoreMesh(core_axis_name="c", subcore_axis_name="s")

    @functools.partial(
        pl.kernel,
        mesh=mesh,
        compiler_params=pltpu.CompilerParams(needs_layout_passes=False),
        out_type=jax.ShapeDtypeStruct((nw, NSLOTS), jnp.float32),
        scratch_types=[
            pltpu.VMEM((NSLOTS,), jnp.float32),
            pltpu.VMEM((ROWS_PER_CHUNK, COLS // 2), jnp.int32),
            pltpu.VMEM((ROWS_PER_CHUNK, COLS // 2), jnp.int32),
            pltpu.VMEM((ROWS_PER_CHUNK, COLS // 2), jnp.int32),
            pltpu.VMEM((ROWS_PER_CHUNK, COLS // 2), jnp.int32),
            pltpu.SemaphoreType.DMA,
            pltpu.SemaphoreType.DMA,
            pltpu.SemaphoreType.DMA,
            pltpu.SemaphoreType.DMA,
        ],
    )
    def hist_kernel(s_hbm, out_hbm, hist, sb0, sb1, sb2, sb3,
                    sem0, sem1, sem2, sem3):
        wid = lax.axis_index("s") * nc + lax.axis_index("c")
        img = wid // 2
        half = wid % 2
        ones = jnp.ones((NLANES,), jnp.float32)

        def zero_body(i, carry):
            z = jnp.zeros((NLANES,), jnp.float32)
            for k in range(32):
                hist[pl.ds((i * 32 + k) * NLANES, NLANES)] = z
            return carry

        lax.fori_loop(0, NSLOTS // (32 * NLANES), zero_body, 0)

        sbufs = (sb0, sb1, sb2, sb3)
        sems = (sem0, sem1, sem2, sem3)
        nbuf = len(sbufs)

        def start(c):
            row0 = half * HALF + c * ROWS_PER_CHUNK
            return pltpu.async_copy(
                s_hbm.at[img, pl.ds(row0, ROWS_PER_CHUNK), :],
                sbufs[c % nbuf], sems[c % nbuf])

        pending = [start(c) for c in range(nbuf - 1)]
        for c in range(NCHUNKS):
            pending[c % (nbuf - 1)].wait()
            if c + nbuf - 1 < NCHUNKS:
                pending[c % (nbuf - 1)] = start(c + nbuf - 1)
            sb = sbufs[c % nbuf]

            def body(r, carry):
                for g in range(COLS // 2 // NLANES):
                    p = sb[r, pl.ds(g * NLANES, NLANES)]
                    s0 = jnp.bitwise_and(p, 0xFFFF)
                    s1 = jnp.right_shift(p, 16)
                    plsc.addupdate_scatter(hist, [s0], ones)
                    plsc.addupdate_scatter(hist, [s1], ones)
                return carry

            lax.fori_loop(0, ROWS_PER_CHUNK, body, 0)

        pltpu.sync_copy(hist, out_hbm.at[wid])

    return hist_kernel(slots3)


def _cumsum_cols(x, width):
    rows = x.shape[0]
    k = 1
    while k < width:
        shifted = jnp.concatenate(
            [jnp.zeros((rows, k), x.dtype), x[:, : width - k]], axis=1)
        x = x + shifted
        k *= 2
    return x


def _cumsum_rows(x, height):
    cols = x.shape[1]
    k = 1
    while k < height:
        shifted = jnp.concatenate(
            [jnp.zeros((k, cols), x.dtype), x[: height - k, :]], axis=0)
        x = x + shifted
        k *= 2
    return x


def _tc_finalize_body(h_ref, out_ref):
    h = h_ref[...]
    g = jnp.sum(h, axis=0)

    s = g
    for k in (1, 2, 4, 8):
        s = s + jnp.concatenate(
            [s[:, k:], jnp.zeros((HR, k), jnp.float32)], axis=1)

    row = lax.broadcasted_iota(jnp.int32, (HR, 128), 0)
    col = lax.broadcasted_iota(jnp.int32, (HR, 128), 1)
    is_bin = (col % 16) == 0
    cnt = jnp.where(is_bin, s, 0.0)

    neg_rows = row < NEGR
    mneg = jnp.where(neg_rows, cnt, 0.0)
    mpos = jnp.where(neg_rows, 0.0, cnt)
    n_neg = jnp.sum(mneg)
    p_pos = jnp.sum(mpos)
    n_all = n_neg + p_pos

    colcs = _cumsum_cols(mneg, 128)
    rowtot = colcs[:, 127:128]
    rowpre = _cumsum_rows(rowtot, HR) - rowtot
    cum_inc = colcs + rowpre

    r = n_neg - cum_inc
    rpm = r + mneg

    vbin = (row % NEGR) * 8 + col // 16
    xm = XLO + (vbin.astype(jnp.float32) + 0.5) / SCALE
    vm = 1.0 / (1.0 + jnp.exp(-xm))

    denom = jnp.maximum((p_pos + r) * (p_pos + rpm), 1.0)
    dw = mneg * p_pos / denom
    loss_neg = jnp.sum(dw * (1.0 + vm))

    maxv = jnp.max(jnp.where(mneg > 0.0, vm, -1.0))
    loss_neg = jnp.where(
        p_pos > 0.5, loss_neg,
        jnp.where(n_neg > 0.5, 1.0 + maxv, 0.0))

    loss_pos = jnp.sum(mpos * (1.0 - vm)) / jnp.maximum(n_all, 1.0)
    out_ref[...] = jnp.reshape(loss_neg + loss_pos, (1, 1))


def _tc_finalize(hists):
    out = pl.pallas_call(
        _tc_finalize_body,
        out_shape=jax.ShapeDtypeStruct((1, 1), jnp.float32),
    )(hists)
    return out.reshape(())


def kernel(inputs, targets):
    slots = _tc_prepass(inputs, targets)
    hists = _sc_histogram(slots)
    return _tc_finalize(hists.reshape(-1, HR, 128))


# speedup vs baseline: 1.0488x; 1.0488x over previous
import functools

import jax
import jax.numpy as jnp
from jax import lax
from jax.experimental import pallas as pl
from jax.experimental.pallas import tpu as pltpu
from jax.experimental.pallas import tpu_sc as plsc

NBINS = 512
XLO = -9.0
XHI = 9.0
SCALE = NBINS / (XHI - XLO)
NLANES = 16
NSLOTS = 2 * NBINS * NLANES
TSHIFT = (NBINS * NLANES).bit_length() - 1
HR = NSLOTS // 128
NEGR = HR // 2

IMG = 16
ROWS = 512
COLS = 512

ROWS_PER_CHUNK = 32
CHUNK = ROWS_PER_CHUNK * COLS
HALF = ROWS // 2
NCHUNKS = HALF // ROWS_PER_CHUNK


def _tc_prepass_body(x_ref, t_ref, s_ref):
    s16 = SCALE * 16.0
    off16 = -XLO * SCALE * 16.0
    hi16 = float((NBINS - 1) * 16 + 15)
    x = x_ref[...]
    t = t_ref[...]
    u = x * s16 + off16
    u = jnp.minimum(jnp.maximum(u, 0.0), hi16)
    b16 = jnp.bitwise_and(u.astype(jnp.int32), ~15)
    tshift = jnp.left_shift(t.astype(jnp.int32), TSHIFT)
    lane = jnp.bitwise_and(
        lax.broadcasted_iota(jnp.int32, x.shape, x.ndim - 1), 15)
    slots = b16 + tshift + lane
    lo = slots[:, :, : COLS // 2]
    hi = slots[:, :, COLS // 2 :]
    s_ref[...] = jnp.bitwise_or(lo, jnp.left_shift(hi, 16))


PRE_SPLIT = 1
PRE_R = ROWS // PRE_SPLIT


def _tc_prepass(x3, t3):
    return pl.pallas_call(
        _tc_prepass_body,
        grid=(IMG // 2,),
        in_specs=[
            pl.BlockSpec((2, ROWS, COLS), lambda i: (i, 0, 0)),
            pl.BlockSpec((2, ROWS, COLS), lambda i: (i, 0, 0)),
        ],
        out_specs=pl.BlockSpec((2, ROWS, COLS // 2), lambda i: (i, 0, 0)),
        out_shape=jax.ShapeDtypeStruct((IMG, ROWS, COLS // 2), jnp.int32),
    )(x3, t3)


def _sc_histogram(slots3):
    info = plsc.get_sparse_core_info()
    nc, ns = info.num_cores, info.num_subcores
    nw = nc * ns
    mesh = plsc.VectorSubcoreMesh(core_axis_name="c", subcore_axis_name="s")

    @functools.partial(
        pl.kernel,
        mesh=mesh,
        compiler_params=pltpu.CompilerParams(needs_layout_passes=False),
        out_type=jax.ShapeDtypeStruct((nw, NSLOTS), jnp.float32),
        scratch_types=[
            pltpu.VMEM((NSLOTS,), jnp.float32),
            pltpu.VMEM((ROWS_PER_CHUNK, COLS // 2), jnp.int32),
            pltpu.VMEM((ROWS_PER_CHUNK, COLS // 2), jnp.int32),
            pltpu.VMEM((ROWS_PER_CHUNK, COLS // 2), jnp.int32),
            pltpu.VMEM((ROWS_PER_CHUNK, COLS // 2), jnp.int32),
            pltpu.SemaphoreType.DMA,
            pltpu.SemaphoreType.DMA,
            pltpu.SemaphoreType.DMA,
            pltpu.SemaphoreType.DMA,
        ],
    )
    def hist_kernel(s_hbm, out_hbm, hist, sb0, sb1, sb2, sb3,
                    sem0, sem1, sem2, sem3):
        wid = lax.axis_index("s") * nc + lax.axis_index("c")
        img = wid // 2
        half = wid % 2
        ones = jnp.ones((NLANES,), jnp.float32)

        def zero_body(i, carry):
            z = jnp.zeros((NLANES,), jnp.float32)
            for k in range(32):
                hist[pl.ds((i * 32 + k) * NLANES, NLANES)] = z
            return carry

        lax.fori_loop(0, NSLOTS // (32 * NLANES), zero_body, 0)

        sbufs = (sb0, sb1, sb2, sb3)
        sems = (sem0, sem1, sem2, sem3)
        nbuf = len(sbufs)

        def start(c):
            row0 = half * HALF + c * ROWS_PER_CHUNK
            return pltpu.async_copy(
                s_hbm.at[img, pl.ds(row0, ROWS_PER_CHUNK), :],
                sbufs[c % nbuf], sems[c % nbuf])

        pending = [start(c) for c in range(nbuf - 1)]
        for c in range(NCHUNKS):
            pending[c % (nbuf - 1)].wait()
            if c + nbuf - 1 < NCHUNKS:
                pending[c % (nbuf - 1)] = start(c + nbuf - 1)
            sb = sbufs[c % nbuf]

            def body(r, carry):
                for g in range(COLS // 2 // NLANES):
                    p = sb[r, pl.ds(g * NLANES, NLANES)]
                    s0 = jnp.bitwise_and(p, 0xFFFF)
                    s1 = jnp.right_shift(p, 16)
                    plsc.addupdate_scatter(hist, [s0], ones)
                    plsc.addupdate_scatter(hist, [s1], ones)
                return carry

            lax.fori_loop(0, ROWS_PER_CHUNK, body, 0)

        pltpu.sync_copy(hist, out_hbm.at[wid])

    return hist_kernel(slots3)


def _cumsum_cols(x, width):
    rows = x.shape[0]
    k = 1
    while k < width:
        shifted = jnp.concatenate(
            [jnp.zeros((rows, k), x.dtype), x[:, : width - k]], axis=1)
        x = x + shifted
        k *= 2
    return x


def _cumsum_rows(x, height):
    cols = x.shape[1]
    k = 1
    while k < height:
        shifted = jnp.concatenate(
            [jnp.zeros((k, cols), x.dtype), x[: height - k, :]], axis=0)
        x = x + shifted
        k *= 2
    return x


def _tc_finalize_body(h_ref, out_ref):
    h = h_ref[...]
    g = jnp.sum(h, axis=0)

    s = g
    for k in (1, 2, 4, 8):
        s = s + jnp.concatenate(
            [s[:, k:], jnp.zeros((HR, k), jnp.float32)], axis=1)

    row = lax.broadcasted_iota(jnp.int32, (HR, 128), 0)
    col = lax.broadcasted_iota(jnp.int32, (HR, 128), 1)
    is_bin = (col % 16) == 0
    cnt = jnp.where(is_bin, s, 0.0)

    neg_rows = row < NEGR
    mneg = jnp.where(neg_rows, cnt, 0.0)
    mpos = jnp.where(neg_rows, 0.0, cnt)
    n_neg = jnp.sum(mneg)
    p_pos = jnp.sum(mpos)
    n_all = n_neg + p_pos

    colcs = _cumsum_cols(mneg, 128)
    rowtot = colcs[:, 127:128]
    rowpre = _cumsum_rows(rowtot, HR) - rowtot
    cum_inc = colcs + rowpre

    r = n_neg - cum_inc
    rpm = r + mneg

    vbin = (row % NEGR) * 8 + col // 16
    xm = XLO + (vbin.astype(jnp.float32) + 0.5) / SCALE
    vm = 1.0 / (1.0 + jnp.exp(-xm))

    denom = jnp.maximum((p_pos + r) * (p_pos + rpm), 1.0)
    dw = mneg * p_pos / denom
    loss_neg = jnp.sum(dw * (1.0 + vm))

    maxv = jnp.max(jnp.where(mneg > 0.0, vm, -1.0))
    loss_neg = jnp.where(
        p_pos > 0.5, loss_neg,
        jnp.where(n_neg > 0.5, 1.0 + maxv, 0.0))

    loss_pos = jnp.sum(mpos * (1.0 - vm)) / jnp.maximum(n_all, 1.0)
    out_ref[...] = jnp.reshape(loss_neg + loss_pos, (1, 1))


def _tc_finalize(hists):
    out = pl.pallas_call(
        _tc_finalize_body,
        out_shape=jax.ShapeDtypeStruct((1, 1), jnp.float32),
    )(hists)
    return out.reshape(())


def kernel(inputs, targets):
    slots = _tc_prepass(inputs, targets)
    hists = _sc_histogram(slots)
    return _tc_finalize(hists.reshape(-1, HR, 128))


# speedup vs baseline: 1.0595x; 1.0102x over previous
import functools

import jax
import jax.numpy as jnp
from jax import lax
from jax.experimental import pallas as pl
from jax.experimental.pallas import tpu as pltpu
from jax.experimental.pallas import tpu_sc as plsc

NBINS = 512
XLO = -9.0
XHI = 9.0
SCALE = NBINS / (XHI - XLO)
NLANES = 16
NSLOTS = 2 * NBINS * NLANES
TSHIFT = (NBINS * NLANES).bit_length() - 1
HR = NSLOTS // 128
NEGR = HR // 2

IMG = 16
ROWS = 512
COLS = 512

ROWS_PER_CHUNK = 32
CHUNK = ROWS_PER_CHUNK * COLS
HALF = ROWS // 2
NCHUNKS = HALF // ROWS_PER_CHUNK


def _tc_prepass_body(x_ref, t_ref, s_ref):
    s16 = SCALE * 16.0
    off16 = -XLO * SCALE * 16.0
    hi16 = float((NBINS - 1) * 16 + 15)
    x = x_ref[...]
    t = t_ref[...]
    u = x * s16 + off16
    u = jnp.minimum(jnp.maximum(u, 0.0), hi16)
    b16 = jnp.bitwise_and(u.astype(jnp.int32), ~15)
    tshift = jnp.left_shift(t.astype(jnp.int32), TSHIFT)
    lane = jnp.bitwise_and(
        lax.broadcasted_iota(jnp.int32, x.shape, x.ndim - 1), 15)
    slots = b16 + tshift + lane
    lo = slots[:, :, : COLS // 2]
    hi = slots[:, :, COLS // 2 :]
    s_ref[...] = jnp.bitwise_or(lo, jnp.left_shift(hi, 16))


PRE_SPLIT = 1
PRE_R = ROWS // PRE_SPLIT


def _tc_prepass(x3, t3):
    return pl.pallas_call(
        _tc_prepass_body,
        grid=(IMG // 4,),
        in_specs=[
            pl.BlockSpec((4, ROWS, COLS), lambda i: (i, 0, 0)),
            pl.BlockSpec((4, ROWS, COLS), lambda i: (i, 0, 0)),
        ],
        out_specs=pl.BlockSpec((4, ROWS, COLS // 2), lambda i: (i, 0, 0)),
        out_shape=jax.ShapeDtypeStruct((IMG, ROWS, COLS // 2), jnp.int32),
    )(x3, t3)


def _sc_histogram(slots3):
    info = plsc.get_sparse_core_info()
    nc, ns = info.num_cores, info.num_subcores
    nw = nc * ns
    mesh = plsc.VectorSubcoreMesh(core_axis_name="c", subcore_axis_name="s")

    @functools.partial(
        pl.kernel,
        mesh=mesh,
        compiler_params=pltpu.CompilerParams(needs_layout_passes=False),
        out_type=jax.ShapeDtypeStruct((nw, NSLOTS), jnp.float32),
        scratch_types=[
            pltpu.VMEM((NSLOTS,), jnp.float32),
            pltpu.VMEM((ROWS_PER_CHUNK, COLS // 2), jnp.int32),
            pltpu.VMEM((ROWS_PER_CHUNK, COLS // 2), jnp.int32),
            pltpu.VMEM((ROWS_PER_CHUNK, COLS // 2), jnp.int32),
            pltpu.VMEM((ROWS_PER_CHUNK, COLS // 2), jnp.int32),
            pltpu.SemaphoreType.DMA,
            pltpu.SemaphoreType.DMA,
            pltpu.SemaphoreType.DMA,
            pltpu.SemaphoreType.DMA,
        ],
    )
    def hist_kernel(s_hbm, out_hbm, hist, sb0, sb1, sb2, sb3,
                    sem0, sem1, sem2, sem3):
        wid = lax.axis_index("s") * nc + lax.axis_index("c")
        img = wid // 2
        half = wid % 2
        ones = jnp.ones((NLANES,), jnp.float32)

        def zero_body(i, carry):
            z = jnp.zeros((NLANES,), jnp.float32)
            for k in range(32):
                hist[pl.ds((i * 32 + k) * NLANES, NLANES)] = z
            return carry

        lax.fori_loop(0, NSLOTS // (32 * NLANES), zero_body, 0)

        sbufs = (sb0, sb1, sb2, sb3)
        sems = (sem0, sem1, sem2, sem3)
        nbuf = len(sbufs)

        def start(c):
            row0 = half * HALF + c * ROWS_PER_CHUNK
            return pltpu.async_copy(
                s_hbm.at[img, pl.ds(row0, ROWS_PER_CHUNK), :],
                sbufs[c % nbuf], sems[c % nbuf])

        pending = [start(c) for c in range(nbuf - 1)]
        for c in range(NCHUNKS):
            pending[c % (nbuf - 1)].wait()
            if c + nbuf - 1 < NCHUNKS:
                pending[c % (nbuf - 1)] = start(c + nbuf - 1)
            sb = sbufs[c % nbuf]

            def body(r, carry):
                for g in range(COLS // 2 // NLANES):
                    p = sb[r, pl.ds(g * NLANES, NLANES)]
                    s0 = jnp.bitwise_and(p, 0xFFFF)
                    s1 = jnp.right_shift(p, 16)
                    plsc.addupdate_scatter(hist, [s0], ones)
                    plsc.addupdate_scatter(hist, [s1], ones)
                return carry

            lax.fori_loop(0, ROWS_PER_CHUNK, body, 0)

        pltpu.sync_copy(hist, out_hbm.at[wid])

    return hist_kernel(slots3)


def _cumsum_cols(x, width):
    rows = x.shape[0]
    k = 1
    while k < width:
        shifted = jnp.concatenate(
            [jnp.zeros((rows, k), x.dtype), x[:, : width - k]], axis=1)
        x = x + shifted
        k *= 2
    return x


def _cumsum_rows(x, height):
    cols = x.shape[1]
    k = 1
    while k < height:
        shifted = jnp.concatenate(
            [jnp.zeros((k, cols), x.dtype), x[: height - k, :]], axis=0)
        x = x + shifted
        k *= 2
    return x


def _tc_finalize_body(h_ref, out_ref):
    h = h_ref[...]
    g = jnp.sum(h, axis=0)

    s = g
    for k in (1, 2, 4, 8):
        s = s + jnp.concatenate(
            [s[:, k:], jnp.zeros((HR, k), jnp.float32)], axis=1)

    row = lax.broadcasted_iota(jnp.int32, (HR, 128), 0)
    col = lax.broadcasted_iota(jnp.int32, (HR, 128), 1)
    is_bin = (col % 16) == 0
    cnt = jnp.where(is_bin, s, 0.0)

    neg_rows = row < NEGR
    mneg = jnp.where(neg_rows, cnt, 0.0)
    mpos = jnp.where(neg_rows, 0.0, cnt)
    n_neg = jnp.sum(mneg)
    p_pos = jnp.sum(mpos)
    n_all = n_neg + p_pos

    colcs = _cumsum_cols(mneg, 128)
    rowtot = colcs[:, 127:128]
    rowpre = _cumsum_rows(rowtot, HR) - rowtot
    cum_inc = colcs + rowpre

    r = n_neg - cum_inc
    rpm = r + mneg

    vbin = (row % NEGR) * 8 + col // 16
    xm = XLO + (vbin.astype(jnp.float32) + 0.5) / SCALE
    vm = 1.0 / (1.0 + jnp.exp(-xm))

    denom = jnp.maximum((p_pos + r) * (p_pos + rpm), 1.0)
    dw = mneg * p_pos / denom
    loss_neg = jnp.sum(dw * (1.0 + vm))

    maxv = jnp.max(jnp.where(mneg > 0.0, vm, -1.0))
    loss_neg = jnp.where(
        p_pos > 0.5, loss_neg,
        jnp.where(n_neg > 0.5, 1.0 + maxv, 0.0))

    loss_pos = jnp.sum(mpos * (1.0 - vm)) / jnp.maximum(n_all, 1.0)
    out_ref[...] = jnp.reshape(loss_neg + loss_pos, (1, 1))


def _tc_finalize(hists):
    out = pl.pallas_call(
        _tc_finalize_body,
        out_shape=jax.ShapeDtypeStruct((1, 1), jnp.float32),
    )(hists)
    return out.reshape(())


def kernel(inputs, targets):
    slots = _tc_prepass(inputs, targets)
    hists = _sc_histogram(slots)
    return _tc_finalize(hists.reshape(-1, HR, 128))
